# baseline (speedup 1.0000x reference)
import functools

import jax
import jax.numpy as jnp
from jax import lax
from jax.experimental import pallas as pl
from jax.experimental.pallas import tpu as pltpu
from jax.experimental.pallas import tpu_sc as plsc

N = 10000
D = 256
H = 512
HALF = 128
NC = 2
NS = 16
ROWS_PER_TILE = 640
NPAD = NS * ROWS_PER_TILE
K = 80
CH = 128
EPT = K * CH
EPAD = EPT * NS
R = 1000


def _sc_scatter(table, src_idx, dst_idx, zeros2d, zeros1d, with_deg):
    mesh = plsc.VectorSubcoreMesh(core_axis_name="c", subcore_axis_name="s",
                                  num_cores=NC, num_subcores=NS)
    out_type = (jax.ShapeDtypeStruct((NC, NPAD, HALF), jnp.float32),
                jax.ShapeDtypeStruct((NPAD,), jnp.float32))
    NB = 4
    W = 8
    G = CH // W

    def body(table_h, src_h, dst_h, z2_h, z1_h, acc_out, deg_out,
             src_w, dst_w, rows_v, ones_v, acc, dacc, gsems, ssems, dsem, wsem):
        c = lax.axis_index("c")
        s = lax.axis_index("s")
        row0 = s * ROWS_PER_TILE
        pltpu.sync_copy(z2_h, acc.at[pl.ds(row0, ROWS_PER_TILE)])
        if with_deg:
            pltpu.sync_copy(z1_h, dacc.at[pl.ds(row0, ROWS_PER_TILE)])
            for i in range(K // 16):
                ones_v[pl.ds(i * 16, 16)] = jnp.full((16,), 1.0, jnp.float32)
        pltpu.sync_copy(src_h.at[c, s, pl.ds(0, W)], src_w.at[0])
        pltpu.sync_copy(dst_h.at[s, pl.ds(0, W)], dst_w.at[0])
        plsc.subcore_barrier()
        for b in range(NB):
            pltpu.async_copy(table_h.at[src_w.at[0, b]], rows_v.at[b], gsems[b])

        def group(g, carry):
            sel = lax.rem(g, 2)
            nxt = 1 - sel

            @pl.when(g + 1 < G)
            def _():
                pltpu.async_copy(src_h.at[c, s, pl.ds((g + 1) * W, W)],
                                 src_w.at[nxt], wsem)
                pltpu.async_copy(dst_h.at[s, pl.ds((g + 1) * W, W)],
                                 dst_w.at[nxt], wsem)

            if with_deg:
                for w in range(W):
                    pltpu.async_copy(ones_v, dacc.at[dst_w.at[sel, w]],
                                     dsem, add=True)
            for w in range(W):
                b = w % NB
                pltpu.make_async_copy(table_h.at[src_w.at[sel, w]],
                                      rows_v.at[b], gsems[b]).wait()
                pltpu.async_copy(rows_v.at[b], acc.at[dst_w.at[sel, w]],
                                 ssems[b], add=True)
                pltpu.make_async_copy(rows_v.at[b], acc.at[dst_w.at[sel, w]],
                                      ssems[b]).wait()
                if w + NB < W:
                    pltpu.async_copy(table_h.at[src_w.at[sel, w + NB]],
                                     rows_v.at[b], gsems[b])
                else:
                    @pl.when(g + 1 < G)
                    def _(w=w, b=b, first=(w + NB == W)):
                        if first:
                            pltpu.make_async_copy(
                                src_h.at[c, s, pl.ds((g + 1) * W, W)],
                                src_w.at[nxt], wsem).wait()
                            pltpu.make_async_copy(
                                dst_h.at[s, pl.ds((g + 1) * W, W)],
                                dst_w.at[nxt], wsem).wait()
                        pltpu.async_copy(table_h.at[src_w.at[nxt, w + NB - W]],
                                         rows_v.at[b], gsems[b])
            if with_deg:
                for w in range(W):
                    pltpu.make_async_copy(ones_v, dacc.at[dst_w.at[sel, w]],
                                          dsem).wait()
            return carry

        lax.fori_loop(0, G, group, 0)
        plsc.subcore_barrier()
        pltpu.sync_copy(acc.at[pl.ds(row0, ROWS_PER_TILE)],
                        acc_out.at[c, pl.ds(row0, ROWS_PER_TILE)])
        if with_deg:
            @pl.when(jnp.logical_and(c == 0, s == 0))
            def _():
                pltpu.sync_copy(dacc, deg_out)
        else:
            @pl.when(jnp.logical_and(c == 0, s == 0))
            def _():
                pltpu.sync_copy(z1_h, deg_out.at[pl.ds(0, ROWS_PER_TILE)])

    kfn = pl.kernel(
        body, out_type=out_type, mesh=mesh,
        scratch_types=[
            pltpu.VMEM((2, W, K), jnp.int32),
            pltpu.VMEM((2, W, K), jnp.int32),
            pltpu.VMEM((NB, K, HALF), jnp.float32),
            pltpu.VMEM((K,), jnp.float32),
            pltpu.VMEM_SHARED((NPAD, HALF), jnp.float32),
            pltpu.VMEM_SHARED((NPAD,), jnp.float32),
            [pltpu.SemaphoreType.DMA] * NB,
            [pltpu.SemaphoreType.DMA] * NB,
            pltpu.SemaphoreType.DMA,
            pltpu.SemaphoreType.DMA,
        ])
    return kfn(table, src_idx, dst_idx, zeros2d, zeros1d)


def _tc_fused(xt, sc1, deg2, W1, b1r, W2):
    def body(x0_r, x1_r, s0_r, s1_r, deg_r, W1_r, b1_r, W2_r, y_r):
        inv = 1.0 / (deg_r[...] + 1.0)
        a0 = (x0_r[0] + s0_r[0]) * inv
        a1 = (x1_r[0] + s1_r[0]) * inv
        a = jnp.concatenate([a0, a1], axis=1)
        h = jnp.dot(a, W1_r[...], preferred_element_type=jnp.float32) + b1_r[...]
        h = jnp.maximum(h, 0.0)
        y = jnp.dot(h, W2_r[...], preferred_element_type=jnp.float32)
        y_r[0] = y[:, :HALF]
        y_r[1] = y[:, HALF:]

    return pl.pallas_call(
        body,
        grid=(N // R,),
        in_specs=[
            pl.BlockSpec((1, R, HALF), lambda i: (0, i, 0)),
            pl.BlockSpec((1, R, HALF), lambda i: (1, i, 0)),
            pl.BlockSpec((1, R, HALF), lambda i: (0, i, 0)),
            pl.BlockSpec((1, R, HALF), lambda i: (1, i, 0)),
            pl.BlockSpec((R, 1), lambda i: (i, 0)),
            pl.BlockSpec((D, H), lambda i: (0, 0)),
            pl.BlockSpec((1, H), lambda i: (0, 0)),
            pl.BlockSpec((H, D), lambda i: (0, 0)),
        ],
        out_specs=pl.BlockSpec((NC, R, HALF), lambda i: (0, i, 0)),
        out_shape=jax.ShapeDtypeStruct((NC, N, HALF), jnp.float32),
    )(xt, xt, sc1, sc1, deg2, W1, b1r, W2)


def _tc_final(sc2, yt, deg2, b2r):
    def body(s0_r, s1_r, y0_r, y1_r, deg_r, b2_r, o_r):
        inv = 1.0 / (deg_r[...] + 1.0)
        p0 = (s0_r[0] + y0_r[0]) * inv
        p1 = (s1_r[0] + y1_r[0]) * inv
        o_r[...] = jnp.concatenate([p0, p1], axis=1) + b2_r[...]

    return pl.pallas_call(
        body,
        grid=(N // R,),
        in_specs=[
            pl.BlockSpec((1, R, HALF), lambda i: (0, i, 0)),
            pl.BlockSpec((1, R, HALF), lambda i: (1, i, 0)),
            pl.BlockSpec((1, R, HALF), lambda i: (0, i, 0)),
            pl.BlockSpec((1, R, HALF), lambda i: (1, i, 0)),
            pl.BlockSpec((R, 1), lambda i: (i, 0)),
            pl.BlockSpec((1, D), lambda i: (0, 0)),
        ],
        out_specs=pl.BlockSpec((R, D), lambda i: (i, 0)),
        out_shape=jax.ShapeDtypeStruct((N, D), jnp.float32),
    )(sc2, sc2, yt, yt, deg2, b2r)


def kernel(x, edge_index, W1, b1, W2, b2):
    src = edge_index[0].astype(jnp.int32)
    dst = edge_index[1].astype(jnp.int32)
    pad = EPAD - src.shape[0]
    srcp = jnp.concatenate([src, jnp.zeros((pad,), jnp.int32)]).reshape(NS, CH, K)
    dstp = jnp.concatenate([dst, jnp.full((pad,), N, jnp.int32)]).reshape(NS, CH, K)
    src2 = jnp.stack([srcp, srcp + N])
    xt = jnp.stack([x[:, :HALF], x[:, HALF:]])
    z2 = jnp.zeros((ROWS_PER_TILE, HALF), jnp.float32)
    z1 = jnp.zeros((ROWS_PER_TILE,), jnp.float32)

    sc1, deg = _sc_scatter(xt.reshape(NC * N, HALF), src2, dstp, z2, z1, True)
    deg2 = deg.reshape(NPAD, 1)
    yt = _tc_fused(xt, sc1, deg2, W1, b1.reshape(1, H), W2)
    sc2, _ = _sc_scatter(yt.reshape(NC * N, HALF), src2, dstp, z2, z1, False)
    return _tc_final(sc2, yt, deg2, b2.reshape(1, D))


# speedup vs baseline: 1.0576x; 1.0576x over previous
import functools

import jax
import jax.numpy as jnp
from jax import lax
from jax.experimental import pallas as pl
from jax.experimental.pallas import tpu as pltpu
from jax.experimental.pallas import tpu_sc as plsc

N = 10000
D = 256
H = 512
HALF = 128
NC = 2
NS = 16
ROWS_PER_TILE = 640
NPAD = NS * ROWS_PER_TILE
K = 128
CH = 80
EPT = K * CH
EPAD = EPT * NS
R = 1000


def _sc_scatter(table, src_idx, dst_idx, zeros2d, zeros1d, with_deg):
    mesh = plsc.VectorSubcoreMesh(core_axis_name="c", subcore_axis_name="s",
                                  num_cores=NC, num_subcores=NS)
    out_type = (jax.ShapeDtypeStruct((NC, NPAD, HALF), jnp.float32),
                jax.ShapeDtypeStruct((NPAD,), jnp.float32))
    NB = 2
    W = 8
    G = CH // W

    def body(table_h, src_h, dst_h, z2_h, z1_h, acc_out, deg_out,
             src_w, dst_w, rows_v, ones_v, acc, dacc, gsems, ssems, dsem, wsem):
        c = lax.axis_index("c")
        s = lax.axis_index("s")
        row0 = s * ROWS_PER_TILE
        pltpu.sync_copy(z2_h, acc.at[pl.ds(row0, ROWS_PER_TILE)])
        if with_deg:
            pltpu.sync_copy(z1_h, dacc.at[pl.ds(row0, ROWS_PER_TILE)])
            for i in range(K // 16):
                ones_v[pl.ds(i * 16, 16)] = jnp.full((16,), 1.0, jnp.float32)
        pltpu.sync_copy(src_h.at[c, s, pl.ds(0, W)], src_w.at[0])
        pltpu.sync_copy(dst_h.at[s, pl.ds(0, W)], dst_w.at[0])
        plsc.subcore_barrier()
        for b in range(NB):
            pltpu.async_copy(table_h.at[src_w.at[0, b]], rows_v.at[b], gsems[b])

        def group(g, carry):
            sel = lax.rem(g, 2)
            nxt = 1 - sel

            @pl.when(g + 1 < G)
            def _():
                pltpu.async_copy(src_h.at[c, s, pl.ds((g + 1) * W, W)],
                                 src_w.at[nxt], wsem)
                pltpu.async_copy(dst_h.at[s, pl.ds((g + 1) * W, W)],
                                 dst_w.at[nxt], wsem)

            if with_deg:
                for w in range(W):
                    pltpu.async_copy(ones_v, dacc.at[dst_w.at[sel, w]],
                                     dsem, add=True)
            for w in range(W):
                b = w % NB
                pltpu.make_async_copy(table_h.at[src_w.at[sel, w]],
                                      rows_v.at[b], gsems[b]).wait()
                pltpu.async_copy(rows_v.at[b], acc.at[dst_w.at[sel, w]],
                                 ssems[b], add=True)
                pltpu.make_async_copy(rows_v.at[b], acc.at[dst_w.at[sel, w]],
                                      ssems[b]).wait()
                if w + NB < W:
                    pltpu.async_copy(table_h.at[src_w.at[sel, w + NB]],
                                     rows_v.at[b], gsems[b])
                else:
                    @pl.when(g + 1 < G)
                    def _(w=w, b=b, first=(w + NB == W)):
                        if first:
                            pltpu.make_async_copy(
                                src_h.at[c, s, pl.ds((g + 1) * W, W)],
                                src_w.at[nxt], wsem).wait()
                            pltpu.make_async_copy(
                                dst_h.at[s, pl.ds((g + 1) * W, W)],
                                dst_w.at[nxt], wsem).wait()
                        pltpu.async_copy(table_h.at[src_w.at[nxt, w + NB - W]],
                                         rows_v.at[b], gsems[b])
            if with_deg:
                for w in range(W):
                    pltpu.make_async_copy(ones_v, dacc.at[dst_w.at[sel, w]],
                                          dsem).wait()
            return carry

        lax.fori_loop(0, G, group, 0)
        plsc.subcore_barrier()
        pltpu.sync_copy(acc.at[pl.ds(row0, ROWS_PER_TILE)],
                        acc_out.at[c, pl.ds(row0, ROWS_PER_TILE)])
        if with_deg:
            @pl.when(jnp.logical_and(c == 0, s == 0))
            def _():
                pltpu.sync_copy(dacc, deg_out)
        else:
            @pl.when(jnp.logical_and(c == 0, s == 0))
            def _():
                pltpu.sync_copy(z1_h, deg_out.at[pl.ds(0, ROWS_PER_TILE)])

    kfn = pl.kernel(
        body, out_type=out_type, mesh=mesh,
        scratch_types=[
            pltpu.VMEM((2, W, K), jnp.int32),
            pltpu.VMEM((2, W, K), jnp.int32),
            pltpu.VMEM((NB, K, HALF), jnp.float32),
            pltpu.VMEM((K,), jnp.float32),
            pltpu.VMEM_SHARED((NPAD, HALF), jnp.float32),
            pltpu.VMEM_SHARED((NPAD,), jnp.float32),
            [pltpu.SemaphoreType.DMA] * NB,
            [pltpu.SemaphoreType.DMA] * NB,
            pltpu.SemaphoreType.DMA,
            pltpu.SemaphoreType.DMA,
        ])
    return kfn(table, src_idx, dst_idx, zeros2d, zeros1d)


def _tc_fused(x, sc1, deg2, W1, b1r, W2):
    def body(x_r, s0_r, s1_r, deg_r, W1_r, b1_r, W2_r, y_r):
        inv = 1.0 / (deg_r[...] + 1.0)
        a0 = (x_r[:, :HALF] + s0_r[0]) * inv
        a1 = (x_r[:, HALF:] + s1_r[0]) * inv
        a = jnp.concatenate([a0, a1], axis=1)
        h = jnp.dot(a, W1_r[...], preferred_element_type=jnp.float32) + b1_r[...]
        h = jnp.maximum(h, 0.0)
        y_r[...] = jnp.dot(h, W2_r[...], preferred_element_type=jnp.float32)

    return pl.pallas_call(
        body,
        grid=(N // R,),
        in_specs=[
            pl.BlockSpec((R, D), lambda i: (i, 0)),
            pl.BlockSpec((1, R, HALF), lambda i: (0, i, 0)),
            pl.BlockSpec((1, R, HALF), lambda i: (1, i, 0)),
            pl.BlockSpec((R, 1), lambda i: (i, 0)),
            pl.BlockSpec((D, H), lambda i: (0, 0)),
            pl.BlockSpec((1, H), lambda i: (0, 0)),
            pl.BlockSpec((H, D), lambda i: (0, 0)),
        ],
        out_specs=pl.BlockSpec((R, D), lambda i: (i, 0)),
        out_shape=jax.ShapeDtypeStruct((N, D), jnp.float32),
    )(x, sc1, sc1, deg2, W1, b1r, W2)


def _tc_final(sc2, yt, deg2, b2r):
    def body(s0_r, s1_r, y_r, deg_r, b2_r, o_r):
        inv = 1.0 / (deg_r[...] + 1.0)
        agg = jnp.concatenate([s0_r[0], s1_r[0]], axis=1)
        o_r[...] = (y_r[...] + agg) * inv + b2_r[...]

    return pl.pallas_call(
        body,
        grid=(N // R,),
        in_specs=[
            pl.BlockSpec((1, R, HALF), lambda i: (0, i, 0)),
            pl.BlockSpec((1, R, HALF), lambda i: (1, i, 0)),
            pl.BlockSpec((R, D), lambda i: (i, 0)),
            pl.BlockSpec((R, 1), lambda i: (i, 0)),
            pl.BlockSpec((1, D), lambda i: (0, 0)),
        ],
        out_specs=pl.BlockSpec((R, D), lambda i: (i, 0)),
        out_shape=jax.ShapeDtypeStruct((N, D), jnp.float32),
    )(sc2, sc2, yt, deg2, b2r)


def kernel(x, edge_index, W1, b1, W2, b2):
    src = edge_index[0].astype(jnp.int32)
    dst = edge_index[1].astype(jnp.int32)
    pad = EPAD - src.shape[0]
    srcp = jnp.concatenate([src, jnp.zeros((pad,), jnp.int32)]).reshape(NS, CH, K)
    dstp = jnp.concatenate([dst, jnp.full((pad,), N, jnp.int32)]).reshape(NS, CH, K)
    src2 = jnp.stack([2 * srcp, 2 * srcp + 1])
    z2 = jnp.zeros((ROWS_PER_TILE, HALF), jnp.float32)
    z1 = jnp.zeros((ROWS_PER_TILE,), jnp.float32)

    sc1, deg = _sc_scatter(x.reshape(NC * N, HALF), src2, dstp, z2, z1, True)
    deg2 = deg.reshape(NPAD, 1)
    yt = _tc_fused(x, sc1, deg2, W1, b1.reshape(1, H), W2)
    sc2, _ = _sc_scatter(yt.reshape(NC * N, HALF), src2, dstp, z2, z1, False)
    return _tc_final(sc2, yt, deg2, b2.reshape(1, D))
